# Initial kernel scaffold; baseline (speedup 1.0000x reference)
import functools

import jax
import jax.numpy as jnp
from jax import lax
from jax.experimental import pallas as pl
from jax.experimental.pallas import tpu as pltpu
from jax.experimental.pallas import tpu_sc as plsc

N_ROWS = 100000
D = 64
B = 4096 * 200
NC, NS = 2, 16
NW = NC * NS
PER_W = B // NW
CHUNK = 512
N_CHUNKS = PER_W // CHUNK


def _make_kernel():
    mesh = plsc.VectorSubcoreMesh(core_axis_name="c", subcore_axis_name="s")

    @functools.partial(
        pl.kernel,
        out_type=jax.ShapeDtypeStruct((B, D), jnp.float32),
        mesh=mesh,
        scratch_types=[
            pltpu.VMEM((CHUNK,), jnp.int32),
            pltpu.VMEM((CHUNK, D), jnp.float32),
            pltpu.SemaphoreType.DMA,
        ],
    )
    def gather_kernel(idx_hbm, table_hbm, out_hbm, idx_v, rows_v, sem):
        wid = lax.axis_index("s") * NC + lax.axis_index("c")
        w_base = wid * PER_W

        def body(i, carry):
            base = w_base + i * CHUNK
            pltpu.sync_copy(idx_hbm.at[pl.ds(base, CHUNK)], idx_v)
            pltpu.async_copy(table_hbm.at[idx_v], rows_v, sem).wait()
            pltpu.sync_copy(rows_v, out_hbm.at[pl.ds(base, CHUNK)])
            return carry

        lax.fori_loop(0, N_CHUNKS, body, 0)

    return gather_kernel


_gather = _make_kernel()


def kernel(t, pe):
    idx = t.reshape(-1).astype(jnp.int32)
    return _gather(idx, pe)


# speedup vs baseline: 4.3120x; 4.3120x over previous
import functools

import jax
import jax.numpy as jnp
from jax import lax
from jax.experimental import pallas as pl
from jax.experimental.pallas import tpu as pltpu
from jax.experimental.pallas import tpu_sc as plsc

N_ROWS = 100000
D = 64
B = 4096 * 200
NC, NS = 2, 16
NW = NC * NS
PER_W = B // NW
CHUNK = 512
N_CHUNKS = PER_W // CHUNK


def _make_kernel():
    mesh = plsc.VectorSubcoreMesh(core_axis_name="c", subcore_axis_name="s")

    @functools.partial(
        pl.kernel,
        out_type=jax.ShapeDtypeStruct((B, D), jnp.float32),
        mesh=mesh,
        scratch_types=[
            pltpu.VMEM((CHUNK,), jnp.int32),
            pltpu.VMEM((CHUNK, D), jnp.float32),
            pltpu.SemaphoreType.DMA,
        ],
        compiler_params=pltpu.CompilerParams(use_tc_tiling_on_sc=False),
    )
    def gather_kernel(idx_hbm, table_hbm, out_hbm, idx_v, rows_v, sem):
        wid = lax.axis_index("s") * NC + lax.axis_index("c")
        w_base = wid * PER_W

        def body(i, carry):
            base = w_base + i * CHUNK
            pltpu.sync_copy(idx_hbm.at[pl.ds(base, CHUNK)], idx_v)
            pltpu.async_copy(table_hbm.at[idx_v], rows_v, sem).wait()
            pltpu.sync_copy(rows_v, out_hbm.at[pl.ds(base, CHUNK)])
            return carry

        lax.fori_loop(0, N_CHUNKS, body, 0)

    return gather_kernel


_gather = _make_kernel()


def kernel(t, pe):
    idx = t.reshape(-1).astype(jnp.int32)
    return _gather(idx, pe)


# speedup vs baseline: 4.6356x; 1.0750x over previous
import functools

import jax
import jax.numpy as jnp
from jax import lax
from jax.experimental import pallas as pl
from jax.experimental.pallas import tpu as pltpu
from jax.experimental.pallas import tpu_sc as plsc

N_ROWS = 100000
D = 64
B = 4096 * 200
NC, NS = 2, 16
NW = NC * NS
PER_W = B // NW
CHUNK = 400
NBUF = 4
N_CHUNKS = PER_W // CHUNK
assert PER_W % CHUNK == 0 and N_CHUNKS % NBUF == 0


def _make_kernel():
    mesh = plsc.VectorSubcoreMesh(core_axis_name="c", subcore_axis_name="s")

    @functools.partial(
        pl.kernel,
        out_type=jax.ShapeDtypeStruct((B, D), jnp.float32),
        mesh=mesh,
        scratch_types=(
            [pltpu.VMEM((NBUF, CHUNK), jnp.int32),
             pltpu.VMEM((NBUF, CHUNK, D), jnp.float32)]
            + [pltpu.SemaphoreType.DMA] * (3 * NBUF)
        ),
        compiler_params=pltpu.CompilerParams(use_tc_tiling_on_sc=False),
    )
    def gather_kernel(idx_hbm, table_hbm, out_hbm, idx_v, rows_v, *sems):
        isem = sems[0:NBUF]
        gsem = sems[NBUF:2 * NBUF]
        osem = sems[2 * NBUF:3 * NBUF]
        wid = lax.axis_index("s") * NC + lax.axis_index("c")
        w_base = wid * PER_W

        def start_idx_load(c, b):
            pltpu.async_copy(idx_hbm.at[pl.ds(w_base + c * CHUNK, CHUNK)],
                             idx_v.at[b], isem[b])

        def start_gather(b):
            pltpu.async_copy(table_hbm.at[idx_v.at[b]], rows_v.at[b], gsem[b])

        def wait_gather(b):
            pltpu.make_async_copy(table_hbm.at[idx_v.at[b]], rows_v.at[b],
                                  gsem[b]).wait()

        def start_store(c, b):
            pltpu.async_copy(rows_v.at[b],
                             out_hbm.at[pl.ds(w_base + c * CHUNK, CHUNK)],
                             osem[b])

        def wait_store(b):
            pltpu.make_async_copy(rows_v.at[b],
                                  out_hbm.at[pl.ds(w_base, CHUNK)],
                                  osem[b]).wait()

        def wait_idx_load(b):
            pltpu.make_async_copy(idx_hbm.at[pl.ds(w_base, CHUNK)],
                                  idx_v.at[b], isem[b]).wait()

        for b in range(NBUF):
            start_idx_load(b, b)

        def body(g, carry):
            for b in range(NBUF):
                i = g * NBUF + b
                j = i - (NBUF - 1)
                sj = (b + 1) % NBUF

                @pl.when(j >= 0)
                def _():
                    wait_gather(sj)
                    start_store(j, sj)

                    @pl.when(j + NBUF < N_CHUNKS)
                    def _():
                        start_idx_load(j + NBUF, sj)

                @pl.when(i >= NBUF)
                def _():
                    wait_store(b)

                wait_idx_load(b)
                start_gather(b)
            return carry

        lax.fori_loop(0, N_CHUNKS // NBUF, body, 0)

        for j in range(N_CHUNKS - NBUF + 1, N_CHUNKS):
            wait_gather(j % NBUF)
            start_store(j, j % NBUF)
        for j in range(N_CHUNKS - NBUF, N_CHUNKS):
            wait_store(j % NBUF)

    return gather_kernel


_gather = _make_kernel()


def kernel(t, pe):
    idx = t.reshape(-1).astype(jnp.int32)
    return _gather(idx, pe)
